# Initial kernel scaffold; baseline (speedup 1.0000x reference)
import jax
import jax.numpy as jnp
from jax.experimental import pallas as pl


def kernel(sequence, tables):
    raise NotImplementedError("write your pallas kernel here")


# speedup vs baseline: 4.3078x; 4.3078x over previous
import numpy as np

import jax
import jax.numpy as jnp
from jax import lax
from jax.experimental import pallas as pl
from jax.experimental.pallas import tpu as pltpu
from jax.experimental.pallas import tpu_sc as plsc

_Q = 8
_V = 1000
_D = 64
_B = 1024
_T = 50

_TOTAL = _B * _Q * _T
_NC = 2
_NS = 16
_NW = _NC * _NS
_PER_W = _TOTAL // _NW
_CHUNK = 128
_NCHUNK = _PER_W // _CHUNK
_LANES = 16
_PERIOD = _Q * _T
_INNER = 25
_LUT_LEN = _PERIOD + _CHUNK

_LUT = np.tile(((np.arange(_PERIOD) // _T) % _Q) * _V, 2)[:_LUT_LEN].astype(
    np.int32)


def _sc_body(seq_hbm, lut_hbm, tab_hbm, out_hbm, lut_v, idx_v, rows_v, gsem):
    wid = lax.axis_index("s") * _NC + lax.axis_index("c")
    base_w = wid * _PER_W
    pltpu.sync_copy(lut_hbm, lut_v)

    def super_chunk(s, carry):
        sbase = base_w + s * (_CHUNK * _INNER)
        for j2 in range(_INNER):
            base = pl.multiple_of(sbase + j2 * _CHUNK, _CHUNK)
            phase = (j2 * _CHUNK) % _PERIOD
            pltpu.sync_copy(seq_hbm.at[pl.ds(base, _CHUNK)], idx_v)
            for j in range(_CHUNK // _LANES):
                sl = pl.ds(j * _LANES, _LANES)
                idx_v[sl] = idx_v[sl] + lut_v[pl.ds(phase + j * _LANES, _LANES)]
            pltpu.async_copy(tab_hbm.at[idx_v], rows_v, gsem).wait()
            pltpu.sync_copy(rows_v, out_hbm.at[pl.ds(base, _CHUNK)])
        return carry

    lax.fori_loop(0, _NCHUNK // _INNER, super_chunk, 0)


@jax.jit
def kernel(sequence, tables):
    seq_flat = sequence.reshape(-1).astype(jnp.int32)
    tab_flat = tables.reshape(_Q * _V, _D)
    lut = jnp.asarray(_LUT)
    mesh = plsc.VectorSubcoreMesh(core_axis_name="c", subcore_axis_name="s")
    call = pl.kernel(
        _sc_body,
        mesh=mesh,
        out_type=jax.ShapeDtypeStruct((_TOTAL, _D), jnp.float32),
        scratch_types=[
            pltpu.VMEM((_LUT_LEN,), jnp.int32),
            pltpu.VMEM((_CHUNK,), jnp.int32),
            pltpu.VMEM((_CHUNK, _D), jnp.float32),
            pltpu.SemaphoreType.DMA,
        ],
        compiler_params=pltpu.CompilerParams(use_tc_tiling_on_sc=False),
    )
    out = call(seq_flat, lut, tab_flat)
    return out.reshape(_B, _Q, _T, _D)


# speedup vs baseline: 5.4606x; 1.2676x over previous
import numpy as np

import jax
import jax.numpy as jnp
from jax import lax
from jax.experimental import pallas as pl
from jax.experimental.pallas import tpu as pltpu
from jax.experimental.pallas import tpu_sc as plsc

_Q = 8
_V = 1000
_D = 64
_B = 1024
_T = 50

_TOTAL = _B * _Q * _T
_NC = 2
_NS = 16
_NW = _NC * _NS
_PER_W = _TOTAL // _NW
_CHUNK = 128
_NCHUNK = _PER_W // _CHUNK
_LANES = 16
_PERIOD = _Q * _T
_LUT_LEN = _PERIOD + _CHUNK

_LUT = np.tile(((np.arange(_PERIOD) // _T) % _Q) * _V, 2)[:_LUT_LEN].astype(
    np.int32)


def _sc_body(seq_hbm, lut_hbm, tab_hbm, out_hbm,
             lut_v, idx0, idx1, idx2, rows0, rows1,
             sem_i0, sem_i1, sem_i2, sem_g0, sem_g1, sem_o0, sem_o1):
    idx = (idx0, idx1, idx2)
    rows = (rows0, rows1)
    sem_i = (sem_i0, sem_i1, sem_i2)
    sem_g = (sem_g0, sem_g1)
    sem_o = (sem_o0, sem_o1)

    wid = lax.axis_index("s") * _NC + lax.axis_index("c")
    base_w = wid * _PER_W
    pltpu.sync_copy(lut_hbm, lut_v)

    def seq_slice(g):
        return seq_hbm.at[pl.ds(pl.multiple_of(base_w + g * _CHUNK, _CHUNK),
                                _CHUNK)]

    def out_slice(g):
        return out_hbm.at[pl.ds(pl.multiple_of(base_w + g * _CHUNK, _CHUNK),
                                _CHUNK)]

    h_i = [None] * _NCHUNK
    h_g = [None] * _NCHUNK
    h_o = [None] * _NCHUNK
    h_i[0] = pltpu.async_copy(seq_slice(0), idx[0], sem_i[0])
    h_i[1] = pltpu.async_copy(seq_slice(1), idx[1], sem_i[1])

    for g in range(_NCHUNK):
        b = g & 1
        i = g % 3
        h_i[g].wait()
        phase = (g * _CHUNK) % _PERIOD
        for j in range(_CHUNK // _LANES):
            sl = pl.ds(j * _LANES, _LANES)
            idx[i][sl] = idx[i][sl] + lut_v[pl.ds(phase + j * _LANES, _LANES)]
        if g >= 2:
            h_o[g - 2].wait()
        h_g[g] = pltpu.async_copy(tab_hbm.at[idx[i]], rows[b], sem_g[b])
        if g >= 1:
            h_g[g - 1].wait()
            h_o[g - 1] = pltpu.async_copy(rows[1 - b], out_slice(g - 1),
                                          sem_o[1 - b])
        if g + 2 < _NCHUNK:
            h_i[g + 2] = pltpu.async_copy(seq_slice(g + 2), idx[(g + 2) % 3],
                                          sem_i[(g + 2) % 3])

    g_last = _NCHUNK - 1
    h_g[g_last].wait()
    h_o[g_last] = pltpu.async_copy(rows[g_last & 1], out_slice(g_last),
                                   sem_o[g_last & 1])
    h_o[g_last - 1].wait()
    h_o[g_last].wait()


@jax.jit
def kernel(sequence, tables):
    seq_flat = sequence.reshape(-1).astype(jnp.int32)
    tab_flat = tables.reshape(_Q * _V, _D)
    lut = jnp.asarray(_LUT)
    mesh = plsc.VectorSubcoreMesh(core_axis_name="c", subcore_axis_name="s")
    call = pl.kernel(
        _sc_body,
        mesh=mesh,
        out_type=jax.ShapeDtypeStruct((_TOTAL, _D), jnp.float32),
        scratch_types=[
            pltpu.VMEM((_LUT_LEN,), jnp.int32),
            pltpu.VMEM((_CHUNK,), jnp.int32),
            pltpu.VMEM((_CHUNK,), jnp.int32),
            pltpu.VMEM((_CHUNK,), jnp.int32),
            pltpu.VMEM((_CHUNK, _D), jnp.float32),
            pltpu.VMEM((_CHUNK, _D), jnp.float32),
            pltpu.SemaphoreType.DMA,
            pltpu.SemaphoreType.DMA,
            pltpu.SemaphoreType.DMA,
            pltpu.SemaphoreType.DMA,
            pltpu.SemaphoreType.DMA,
            pltpu.SemaphoreType.DMA,
            pltpu.SemaphoreType.DMA,
        ],
        compiler_params=pltpu.CompilerParams(use_tc_tiling_on_sc=False),
    )
    out = call(seq_flat, lut, tab_flat)
    return out.reshape(_B, _Q, _T, _D)


# speedup vs baseline: 6.1934x; 1.1342x over previous
import numpy as np

import jax
import jax.numpy as jnp
from jax import lax
from jax.experimental import pallas as pl
from jax.experimental.pallas import tpu as pltpu
from jax.experimental.pallas import tpu_sc as plsc

_Q = 8
_V = 1000
_D = 64
_B = 1024
_T = 50

_TOTAL = _B * _Q * _T
_NC = 2
_NS = 16
_NW = _NC * _NS
_PER_W = _TOTAL // _NW
_CHUNK = 128
_NCHUNK = _PER_W // _CHUNK
_LANES = 16
_PERIOD = _Q * _T
_LUT_LEN = _PERIOD + _CHUNK

_LUT = np.tile(((np.arange(_PERIOD) // _T) % _Q) * _V, 2)[:_LUT_LEN].astype(
    np.int32)


def _sc_body(seq_hbm, lut_hbm, tab_hbm, out_hbm,
             tab_sh, lut_v, idx0, idx1, idx2, rows0, rows1,
             sem_i0, sem_i1, sem_i2, sem_g0, sem_g1, sem_o0, sem_o1):
    idx = (idx0, idx1, idx2)
    rows = (rows0, rows1)
    sem_i = (sem_i0, sem_i1, sem_i2)
    sem_g = (sem_g0, sem_g1)
    sem_o = (sem_o0, sem_o1)

    wid = lax.axis_index("s") * _NC + lax.axis_index("c")
    base_w = wid * _PER_W
    sid = lax.axis_index("s")
    rows_per_tile = (_Q * _V) // _NS
    tb = pl.multiple_of(sid * rows_per_tile, rows_per_tile)
    pltpu.sync_copy(tab_hbm.at[pl.ds(tb, rows_per_tile)],
                    tab_sh.at[pl.ds(tb, rows_per_tile)])
    pltpu.sync_copy(lut_hbm, lut_v)
    plsc.subcore_barrier()

    def seq_slice(g):
        return seq_hbm.at[pl.ds(pl.multiple_of(base_w + g * _CHUNK, _CHUNK),
                                _CHUNK)]

    def out_slice(g):
        return out_hbm.at[pl.ds(pl.multiple_of(base_w + g * _CHUNK, _CHUNK),
                                _CHUNK)]

    h_i = [None] * _NCHUNK
    h_g = [None] * _NCHUNK
    h_o = [None] * _NCHUNK
    h_i[0] = pltpu.async_copy(seq_slice(0), idx[0], sem_i[0])
    h_i[1] = pltpu.async_copy(seq_slice(1), idx[1], sem_i[1])

    for g in range(_NCHUNK):
        b = g & 1
        i = g % 3
        h_i[g].wait()
        phase = (g * _CHUNK) % _PERIOD
        for j in range(_CHUNK // _LANES):
            sl = pl.ds(j * _LANES, _LANES)
            idx[i][sl] = idx[i][sl] + lut_v[pl.ds(phase + j * _LANES, _LANES)]
        if g >= 2:
            h_o[g - 2].wait()
        h_g[g] = pltpu.async_copy(tab_sh.at[idx[i]], rows[b], sem_g[b])
        if g >= 1:
            h_g[g - 1].wait()
            h_o[g - 1] = pltpu.async_copy(rows[1 - b], out_slice(g - 1),
                                          sem_o[1 - b])
        if g + 2 < _NCHUNK:
            h_i[g + 2] = pltpu.async_copy(seq_slice(g + 2), idx[(g + 2) % 3],
                                          sem_i[(g + 2) % 3])

    g_last = _NCHUNK - 1
    h_g[g_last].wait()
    h_o[g_last] = pltpu.async_copy(rows[g_last & 1], out_slice(g_last),
                                   sem_o[g_last & 1])
    h_o[g_last - 1].wait()
    h_o[g_last].wait()


@jax.jit
def kernel(sequence, tables):
    seq_flat = sequence.reshape(-1).astype(jnp.int32)
    tab_flat = tables.reshape(_Q * _V, _D)
    lut = jnp.asarray(_LUT)
    mesh = plsc.VectorSubcoreMesh(core_axis_name="c", subcore_axis_name="s")
    call = pl.kernel(
        _sc_body,
        mesh=mesh,
        out_type=jax.ShapeDtypeStruct((_TOTAL, _D), jnp.float32),
        scratch_types=[
            pltpu.VMEM_SHARED((_Q * _V, _D), jnp.float32),
            pltpu.VMEM((_LUT_LEN,), jnp.int32),
            pltpu.VMEM((_CHUNK,), jnp.int32),
            pltpu.VMEM((_CHUNK,), jnp.int32),
            pltpu.VMEM((_CHUNK,), jnp.int32),
            pltpu.VMEM((_CHUNK, _D), jnp.float32),
            pltpu.VMEM((_CHUNK, _D), jnp.float32),
            pltpu.SemaphoreType.DMA,
            pltpu.SemaphoreType.DMA,
            pltpu.SemaphoreType.DMA,
            pltpu.SemaphoreType.DMA,
            pltpu.SemaphoreType.DMA,
            pltpu.SemaphoreType.DMA,
            pltpu.SemaphoreType.DMA,
        ],
        compiler_params=pltpu.CompilerParams(use_tc_tiling_on_sc=False),
    )
    out = call(seq_flat, lut, tab_flat)
    return out.reshape(_B, _Q, _T, _D)


# speedup vs baseline: 6.2700x; 1.0124x over previous
import numpy as np

import jax
import jax.numpy as jnp
from jax import lax
from jax.experimental import pallas as pl
from jax.experimental.pallas import tpu as pltpu
from jax.experimental.pallas import tpu_sc as plsc

_Q = 8
_V = 1000
_D = 64
_B = 1024
_T = 50

_TOTAL = _B * _Q * _T
_NC = 2
_NS = 16
_NW = _NC * _NS
_PER_W = _TOTAL // _NW
_CHUNK = 128
_NCHUNK = _PER_W // _CHUNK
_LANES = 16
_PERIOD = _Q * _T
_LUT_LEN = _PERIOD + _CHUNK

_LAG = 3
_NROWS = _LAG + 1
_PD = _LAG + 1
_NIDX = _PD + _LAG

_LUT = np.tile(((np.arange(_PERIOD) // _T) % _Q) * _V, 2)[:_LUT_LEN].astype(
    np.int32)


def _sc_body(seq_hbm, lut_hbm, tab_hbm, out_hbm, tab_sh, lut_v, *rest):
    idx = rest[:_NIDX]
    rows = rest[_NIDX:_NIDX + _NROWS]
    sem_i = rest[_NIDX + _NROWS:2 * _NIDX + _NROWS]
    sem_g = rest[2 * _NIDX + _NROWS:2 * _NIDX + 2 * _NROWS]
    sem_o = rest[2 * _NIDX + 2 * _NROWS:2 * _NIDX + 3 * _NROWS]

    wid = lax.axis_index("s") * _NC + lax.axis_index("c")
    base_w = wid * _PER_W
    sid = lax.axis_index("s")
    rows_per_tile = (_Q * _V) // _NS
    tb = pl.multiple_of(sid * rows_per_tile, rows_per_tile)
    pltpu.sync_copy(tab_hbm.at[pl.ds(tb, rows_per_tile)],
                    tab_sh.at[pl.ds(tb, rows_per_tile)])
    pltpu.sync_copy(lut_hbm, lut_v)
    plsc.subcore_barrier()

    def seq_slice(g):
        return seq_hbm.at[pl.ds(pl.multiple_of(base_w + g * _CHUNK, _CHUNK),
                                _CHUNK)]

    def out_slice(g):
        return out_hbm.at[pl.ds(pl.multiple_of(base_w + g * _CHUNK, _CHUNK),
                                _CHUNK)]

    h_i = [None] * _NCHUNK
    h_g = [None] * _NCHUNK
    h_o = [None] * _NCHUNK
    for g0 in range(_PD):
        h_i[g0] = pltpu.async_copy(seq_slice(g0), idx[g0 % _NIDX],
                                   sem_i[g0 % _NIDX])

    def writeback(g):
        h_g[g].wait()
        h_o[g] = pltpu.async_copy(rows[g % _NROWS], out_slice(g),
                                  sem_o[g % _NROWS])

    for g in range(_NCHUNK):
        i = g % _NIDX
        h_i[g].wait()
        phase = (g * _CHUNK) % _PERIOD
        for j in range(_CHUNK // _LANES):
            sl = pl.ds(j * _LANES, _LANES)
            idx[i][sl] = idx[i][sl] + lut_v[pl.ds(phase + j * _LANES, _LANES)]
        if g - _NROWS >= 0:
            h_o[g - _NROWS].wait()
        h_g[g] = pltpu.async_copy(tab_sh.at[idx[i]], rows[g % _NROWS],
                                  sem_g[g % _NROWS])
        if g - _LAG >= 0:
            writeback(g - _LAG)
        if g + _PD < _NCHUNK:
            h_i[g + _PD] = pltpu.async_copy(seq_slice(g + _PD),
                                            idx[(g + _PD) % _NIDX],
                                            sem_i[(g + _PD) % _NIDX])

    for g in range(_NCHUNK - _LAG, _NCHUNK):
        writeback(g)
    for g in range(_NCHUNK - _NROWS, _NCHUNK):
        h_o[g].wait()


@jax.jit
def kernel(sequence, tables):
    seq_flat = sequence.reshape(-1).astype(jnp.int32)
    tab_flat = tables.reshape(_Q * _V, _D)
    lut = jnp.asarray(_LUT)
    mesh = plsc.VectorSubcoreMesh(core_axis_name="c", subcore_axis_name="s")
    scratch = [
        pltpu.VMEM_SHARED((_Q * _V, _D), jnp.float32),
        pltpu.VMEM((_LUT_LEN,), jnp.int32),
    ]
    scratch += [pltpu.VMEM((_CHUNK,), jnp.int32) for _ in range(_NIDX)]
    scratch += [pltpu.VMEM((_CHUNK, _D), jnp.float32) for _ in range(_NROWS)]
    scratch += [pltpu.SemaphoreType.DMA] * (_NIDX + 2 * _NROWS)
    call = pl.kernel(
        _sc_body,
        mesh=mesh,
        out_type=jax.ShapeDtypeStruct((_TOTAL, _D), jnp.float32),
        scratch_types=scratch,
        compiler_params=pltpu.CompilerParams(use_tc_tiling_on_sc=False),
    )
    out = call(seq_flat, lut, tab_flat)
    return out.reshape(_B, _Q, _T, _D)
